# baseline (speedup 1.0000x reference)
import functools

import jax
import jax.numpy as jnp
from jax import lax
from jax.experimental import pallas as pl
from jax.experimental.pallas import tpu as pltpu
from jax.experimental.pallas import tpu_sc as plsc

X_DIM = 768
KEY_DIM = 128
N_EXPERTS = 8
T_TOKENS = 8192
NEG = -1e30

_INFO = plsc.get_sparse_core_info()
NC, NS, L = _INFO.num_cores, _INFO.num_subcores, _INFO.num_lanes
NW = NC * NS
TPW = T_TOKENS // NW
GROUPS = TPW // L
CHUNK = 32
NCHUNK = TPW // CHUNK
COLV = X_DIM // L

BT_A = 2048


def _gate_body(gate_ref, w_ref, keysT_ref, scores_ref, wsum_ref):
    q = jnp.dot(gate_ref[...], w_ref[...], preferred_element_type=jnp.float32)
    logits = jnp.dot(q, keysT_ref[...], preferred_element_type=jnp.float32)
    lane = jax.lax.broadcasted_iota(jnp.int32, logits.shape, 1)
    logits = jnp.where(lane < N_EXPERTS, logits, NEG)

    m1 = jnp.max(logits, axis=-1, keepdims=True)
    a1 = jnp.min(jnp.where(logits == m1, lane, KEY_DIM), axis=-1, keepdims=True)
    l2 = jnp.where(lane == a1, NEG, logits)
    m2 = jnp.max(l2, axis=-1, keepdims=True)
    a2 = jnp.min(jnp.where(l2 == m2, lane, KEY_DIM), axis=-1, keepdims=True)

    d = jnp.exp(m2 - m1)
    denom = 1.0 + d
    w1 = 1.0 / denom
    w2 = d / denom

    scores = jnp.where(lane == a1, w1, 0.0) + jnp.where(lane == a2, w2, 0.0)
    scores_ref[...] = scores[:, :N_EXPERTS]
    ones = jnp.full((1, KEY_DIM), 1.0, jnp.float32)
    wsum_ref[...] = lax.dot_general(ones, scores, (((1,), (1,)), ((), ())),
                                    preferred_element_type=jnp.float32)


def _sc_combine(wsum_hbm, raw_hbm, comb_hbm, wbuf, wbc, rbuf0, rbuf1,
                sem_i0, sem_i1, sem_o0, sem_o1):
    wid = lax.axis_index("s") * NC + lax.axis_index("c")
    row0 = wid * TPW
    rbufs = (rbuf0, rbuf1)
    sems_i = (sem_i0, sem_i1)
    sems_o = (sem_o0, sem_o1)

    def chunk_in(c):
        return pltpu.async_copy(
            raw_hbm.at[pl.ds(row0 + c * CHUNK, CHUNK), :], rbufs[c % 2],
            sems_i[c % 2])

    in_handles = [chunk_in(0)]
    pltpu.sync_copy(wsum_hbm.at[:, pl.ds(row0, TPW)], wbuf)

    for g in range(GROUPS):
        wv = wbuf[0, pl.ds(g * L, L)]
        for i in range(L):
            wbc[g * L + i, :] = jnp.full((L,), wv[i])

    out_handles = [None, None]
    for c in range(NCHUNK):
        b = c % 2
        if c + 1 < NCHUNK:
            if out_handles[(c + 1) % 2] is not None:
                out_handles[(c + 1) % 2].wait()
                out_handles[(c + 1) % 2] = None
            in_handles.append(chunk_in(c + 1))
        in_handles[c].wait()
        buf = rbufs[b]

        def body(t, _):
            wv = wbc[c * CHUNK + t, :]
            for j in range(COLV):
                buf[t, pl.ds(j * L, L)] = buf[t, pl.ds(j * L, L)] * wv
            return 0

        lax.fori_loop(0, CHUNK, body, 0)
        out_handles[b] = pltpu.async_copy(
            buf, comb_hbm.at[pl.ds(row0 + c * CHUNK, CHUNK), :], sems_o[b])
    for h in out_handles:
        if h is not None:
            h.wait()


@jax.jit
def kernel(gate_inputs, raw_inputs, W_gate, keys):
    keysT = jnp.zeros((KEY_DIM, KEY_DIM), jnp.float32).at[:, :N_EXPERTS].set(keys.T)
    scores, wsumT = pl.pallas_call(
        _gate_body,
        grid=(T_TOKENS // BT_A,),
        in_specs=[
            pl.BlockSpec((BT_A, X_DIM), lambda i: (i, 0)),
            pl.BlockSpec((X_DIM, KEY_DIM), lambda i: (0, 0)),
            pl.BlockSpec((KEY_DIM, KEY_DIM), lambda i: (0, 0)),
        ],
        out_specs=[
            pl.BlockSpec((BT_A, N_EXPERTS), lambda i: (i, 0)),
            pl.BlockSpec((1, BT_A), lambda i: (0, i)),
        ],
        out_shape=[
            jax.ShapeDtypeStruct((T_TOKENS, N_EXPERTS), jnp.float32),
            jax.ShapeDtypeStruct((1, T_TOKENS), jnp.float32),
        ],
    )(gate_inputs, W_gate, keysT)

    combine = functools.partial(
        pl.kernel,
        mesh=plsc.VectorSubcoreMesh(core_axis_name="c", subcore_axis_name="s"),
        compiler_params=pltpu.CompilerParams(needs_layout_passes=False),
        out_type=jax.ShapeDtypeStruct((T_TOKENS, X_DIM), jnp.float32),
        scratch_types=[
            pltpu.VMEM((1, TPW), jnp.float32),
            pltpu.VMEM((TPW, L), jnp.float32),
            pltpu.VMEM((CHUNK, X_DIM), jnp.float32),
            pltpu.VMEM((CHUNK, X_DIM), jnp.float32),
            pltpu.SemaphoreType.DMA,
            pltpu.SemaphoreType.DMA,
            pltpu.SemaphoreType.DMA,
            pltpu.SemaphoreType.DMA,
        ],
    )(_sc_combine)
    comb = combine(wsumT, raw_inputs)
    return (comb, scores)
